# baseline (speedup 1.0000x reference)
import jax
import jax.numpy as jnp
from jax import lax
from jax.experimental import pallas as pl
from jax.experimental.pallas import tpu as pltpu
from jax.experimental.pallas import tpu_sc as plsc

N = 50000
E = 800000
HID = 64

NPAD = 50176
EPAD = 802816
Q = 50176
NCH = 392
CH = 128
ROWS_T = 3136
BN = 3584
GRID = 14
EQ = 1568
ECH = 112
BB = 14
NBB = 28
NBUF = 4

_mesh = lambda: plsc.VectorSubcoreMesh(core_axis_name="c", subcore_axis_name="s")
_SC_PARAMS = pltpu.CompilerParams(use_tc_tiling_on_sc=False)


def _pre_body(src_hbm, ones_hbm, zeros_hbm, a0, a1, a2, a3,
              t0, t1, t2, t3, out_hbm, o0, o1, o2, o3,
              idx_v, ones_v, z_v, i0, i1, i2, i3, r0, r1, r2, r3, sem, acc):
    c = lax.axis_index("c")
    s = lax.axis_index("s")
    pltpu.sync_copy(src_hbm.at[pl.ds(s * NCH, NCH)], idx_v)
    pltpu.sync_copy(ones_hbm, ones_v)
    pltpu.sync_copy(zeros_hbm, z_v)
    base = s * ROWS_T
    for k in range(24):
        pltpu.sync_copy(z_v, acc.at[pl.ds(base + k * CH, CH)])
    pltpu.sync_copy(z_v.at[pl.ds(0, 64)], acc.at[pl.ds(base + 24 * CH, 64)])

    wid = s * 2 + c
    ebase = wid * EQ
    idxs = (i0, i1, i2, i3)
    rows = (r0, r1, r2, r3)
    tabs = (t0, t1, t2, t3)
    outs = (o0, o1, o2, o3)
    attrs = (a0, a1, a2, a3)

    def estep(k, carry):
        b = ebase + k * ECH
        for j in range(4):
            pltpu.sync_copy(attrs[j].at[pl.ds(b, ECH)], idxs[j])
            pltpu.async_copy(tabs[j].at[idxs[j]], rows[j], sem).wait()
            pltpu.sync_copy(rows[j], outs[j].at[pl.ds(b, ECH)])
        return carry

    lax.fori_loop(0, 14, estep, 0)

    plsc.subcore_barrier()
    lo = c * 196
    hi = 196 + c * (NCH - 196)

    def step(j, carry):
        pltpu.sync_copy(ones_v, acc.at[idx_v.at[j]], add=True)
        return carry

    lax.fori_loop(lo, hi, step, 0)
    plsc.subcore_barrier()
    pltpu.sync_copy(acc.at[pl.ds(base, ROWS_T)],
                    out_hbm.at[pl.ds(c * NPAD + base, ROWS_T)])


def _prop_body(xs_hbm, comb_hbm, zeros_hbm, out_hbm,
               ib0, ib1, r0, r1, r2, r3, acc,
               is0, is1, s0, s1, s2, s3):
    c = lax.axis_index("c")
    s = lax.axis_index("s")
    rows = (r0, r1, r2, r3)
    sems = (s0, s1, s2, s3)
    pltpu.sync_copy(zeros_hbm, r0)
    base = s * ROWS_T
    for k in range(24):
        pltpu.sync_copy(r0, acc.at[pl.ds(base + k * CH, CH)])
    pltpu.sync_copy(r0.at[pl.ds(0, 64)], acc.at[pl.ds(base + 24 * CH, 64)])
    plsc.subcore_barrier()
    cbase = (c * 16 + s) * NBB

    pltpu.async_copy(comb_hbm.at[cbase], ib0, is0)

    def inner(bb, cur, nxt, cursem, nxtsem):
        pltpu.make_async_copy(comb_hbm.at[cbase + bb], cur, cursem).wait()

        @pl.when(bb + 1 < NBB)
        def _():
            pltpu.async_copy(comb_hbm.at[cbase + bb + 1], nxt, nxtsem)

        hs = [None] * NBUF
        for j in range(NBUF):
            hs[j] = pltpu.async_copy(xs_hbm.at[cur.at[j]], rows[j], sems[j])
        for j in range(BB):
            hs[j % NBUF].wait()
            pltpu.sync_copy(rows[j % NBUF], acc.at[cur.at[BB + j]], add=True)
            nxt_j = j + NBUF
            if nxt_j < BB:
                hs[nxt_j % NBUF] = pltpu.async_copy(
                    xs_hbm.at[cur.at[nxt_j]], rows[nxt_j % NBUF],
                    sems[nxt_j % NBUF])

    def blk(bb, carry):
        @pl.when(lax.rem(bb, 2) == 0)
        def _():
            inner(bb, ib0, ib1, is0, is1)

        @pl.when(lax.rem(bb, 2) == 1)
        def _():
            inner(bb, ib1, ib0, is1, is0)

        return carry

    lax.fori_loop(0, NBB, blk, 0)
    plsc.subcore_barrier()
    pltpu.sync_copy(acc.at[pl.ds(base, ROWS_T)],
                    out_hbm.at[pl.ds(c * NPAD + base, ROWS_T)])


def _make_pre():
    return pl.kernel(
        _pre_body,
        out_type=[jax.ShapeDtypeStruct((2 * NPAD, 8), jnp.float32)]
        + [jax.ShapeDtypeStruct((NPAD, 8), jnp.float32)] * 4,
        mesh=_mesh(),
        scratch_types=[
            pltpu.VMEM((NCH, CH), jnp.int32),
            pltpu.VMEM((CH, 8), jnp.float32),
            pltpu.VMEM((CH, 8), jnp.float32),
        ]
        + [pltpu.VMEM((ECH,), jnp.int32)] * 4
        + [pltpu.VMEM((ECH, 8), jnp.float32)] * 4
        + [pltpu.SemaphoreType.DMA]
        + [pltpu.VMEM_SHARED((NPAD, 8), jnp.float32)],
        compiler_params=_SC_PARAMS,
    )


def _make_prop():
    return pl.kernel(
        _prop_body,
        out_type=jax.ShapeDtypeStruct((2 * NPAD, 32), jnp.bfloat16),
        mesh=_mesh(),
        scratch_types=[
            pltpu.VMEM((2 * BB, CH), jnp.int32),
            pltpu.VMEM((2 * BB, CH), jnp.int32),
        ]
        + [pltpu.VMEM((CH, 32), jnp.bfloat16)] * NBUF
        + [pltpu.VMEM_SHARED((NPAD, 32), jnp.bfloat16)]
        + [pltpu.SemaphoreType.DMA] * (2 + NBUF),
        compiler_params=_SC_PARAMS,
    )


def _plane_lo():
    return pl.BlockSpec((BN, 32), lambda i: (i, 0))


def _plane_hi():
    return pl.BlockSpec((BN, 32), lambda i: (i + GRID, 0))


def _row_spec(cols):
    return pl.BlockSpec((BN, cols), lambda i: (i, 0))


def _dlo():
    return pl.BlockSpec((BN, 8), lambda i: (i, 0))


def _dhi():
    return pl.BlockSpec((BN, 8), lambda i: (i + GRID, 0))


def _full_spec(shape):
    nd = len(shape)
    return pl.BlockSpec(shape, lambda i: (0,) * nd)


def _stage_a_body(dlo, dhi, f0, f1, f2, f3, vis, xs_o, x_o, dinv8_o):
    deg = dlo[:, 0:1] + dhi[:, 0:1]
    dinv = jnp.where(deg > 0, lax.rsqrt(jnp.where(deg > 0, deg, 1.0)), 0.0)
    x = jnp.concatenate([f0[...], f1[...], f2[...], f3[...], vis[...]], axis=1)
    x_o[...] = x
    xs_o[...] = (dinv * x).astype(jnp.bfloat16)
    dinv8_o[...] = jnp.broadcast_to(dinv, (BN, 8))


def _stage_bd_body(slo, shi, dinv8, x, W, out_o, ts_o):
    dinv = dinv8[:, 0:1]
    t1 = -dinv * jnp.concatenate([slo[...], shi[...]], axis=1).astype(jnp.float32)
    out_o[...] = (jnp.dot(x[...], W[0], preferred_element_type=jnp.float32)
                  + jnp.dot(t1, W[1], preferred_element_type=jnp.float32))
    ts_o[...] = (dinv * t1).astype(jnp.bfloat16)


def _stage_c_body(slo, shi, dinv8, x, out1, W, b, h_o, hs_o):
    dinv = dinv8[:, 0:1]
    t2 = (-2.0 * dinv * jnp.concatenate([slo[...], shi[...]], axis=1).astype(jnp.float32) - x[...])
    h = jnp.maximum(
        out1[...] + jnp.dot(t2, W[2], preferred_element_type=jnp.float32)
        + b[...], 0.0)
    h_o[...] = h
    hs_o[...] = (dinv * h).astype(jnp.bfloat16)


def _stage_e_body(slo, shi, dinv8, h, out2, W, b, Wp1, bp1, Wp2, bp2,
                  h2_o, z_o):
    dinv = dinv8[:, 0:1]
    t2 = (-2.0 * dinv * jnp.concatenate([slo[...], shi[...]], axis=1).astype(jnp.float32) - h[...])
    h2 = jnp.maximum(
        out2[...] + jnp.dot(t2, W[2], preferred_element_type=jnp.float32)
        + b[...], 0.0)
    h2_o[...] = h2
    a = jnp.dot(h2, Wp1[...], preferred_element_type=jnp.float32) + bp1[...]
    a = jnp.where(a > 0, a, jnp.exp(jnp.minimum(a, 0.0)) - 1.0)
    z_o[...] = jnp.dot(a, Wp2[...], preferred_element_type=jnp.float32) + bp2[...]


def _tc_call(body, in_specs, out_shapes, out_specs):
    return pl.pallas_call(
        body,
        grid=(GRID,),
        in_specs=in_specs,
        out_specs=out_specs,
        out_shape=out_shapes,
    )


@jax.jit
def kernel(edge_index, node_attr, node_vis_feat, len_emb, id_emb, lng_emb,
           lat_emb, W1, b1, W2, b2, Wp1, bp1, Wp2, bp2):
    f32 = jnp.float32
    i32 = jnp.int32

    src = edge_index[0].astype(i32)
    dst = edge_index[1].astype(i32)
    padE = EPAD - E
    srcp = jnp.concatenate([src, jnp.full((padE,), N, i32)])
    dstp = jnp.concatenate([dst, jnp.full((padE,), N, i32)])
    src_plain = srcp.reshape(16 * NCH, CH)
    srco_int = jnp.concatenate([2 * srcp, 2 * srcp + 1]).reshape(2 * 16 * NCH, CH)
    srcb = srco_int.reshape(2, 16, NBB, BB, CH)
    dstb = jnp.broadcast_to(dstp.reshape(1, 16, NBB, BB, CH),
                            (2, 16, NBB, BB, CH))
    comb = jnp.concatenate([srcb, dstb], axis=3).reshape(
        2 * 16 * NBB, 2 * BB, CH)

    attr = jnp.pad(node_attr.astype(i32), ((0, NPAD - N), (0, 0)))
    vis_p = jnp.pad(node_vis_feat.astype(f32), ((0, NPAD - N), (0, 0)))

    zeros8 = jnp.zeros((CH, 8), f32)
    ones8 = jnp.ones((CH, 8), f32)
    zeros32 = jnp.zeros((CH, 32), jnp.bfloat16)

    deg2, f_id, f_len, f_lng, f_lat = _make_pre()(
        src_plain, ones8, zeros8,
        attr[:, 1], attr[:, 0], attr[:, 2], attr[:, 3],
        id_emb.astype(f32), len_emb.astype(f32),
        lng_emb.astype(f32), lat_emb.astype(f32))

    prop = _make_prop()

    b1r = b1.reshape(1, HID).astype(f32)
    b2r = b2.reshape(1, HID).astype(f32)
    bp1r = bp1.reshape(1, 64).astype(f32)
    bp2r = bp2.reshape(1, 32).astype(f32)
    W1f = W1.astype(f32)
    W2f = W2.astype(f32)

    wspec = _full_spec((3, HID, HID))
    bspec = _full_spec((1, HID))

    xs, x64, dinv8 = _tc_call(
        _stage_a_body,
        [_dlo(), _dhi(), _dlo(), _dlo(), _dlo(), _dlo(), _row_spec(32)],
        [jax.ShapeDtypeStruct((NPAD, HID), jnp.bfloat16),
         jax.ShapeDtypeStruct((NPAD, HID), f32),
         jax.ShapeDtypeStruct((NPAD, 8), f32)],
        [_row_spec(HID), _row_spec(HID), _dlo()],
    )(deg2, deg2, f_id, f_len, f_lng, f_lat, vis_p)

    s1 = prop(xs.reshape(2 * NPAD, 32), comb, zeros32)
    out1, t1s = _tc_call(
        _stage_bd_body,
        [_plane_lo(), _plane_hi(), _dlo(), _row_spec(HID), wspec],
        [jax.ShapeDtypeStruct((NPAD, HID), f32),
         jax.ShapeDtypeStruct((NPAD, HID), jnp.bfloat16)],
        [_row_spec(HID), _row_spec(HID)],
    )(s1, s1, dinv8, x64, W1f)

    s2 = prop(t1s.reshape(2 * NPAD, 32), comb, zeros32)
    h1, hs = _tc_call(
        _stage_c_body,
        [_plane_lo(), _plane_hi(), _dlo(), _row_spec(HID), _row_spec(HID),
         wspec, bspec],
        [jax.ShapeDtypeStruct((NPAD, HID), f32),
         jax.ShapeDtypeStruct((NPAD, HID), jnp.bfloat16)],
        [_row_spec(HID), _row_spec(HID)],
    )(s2, s2, dinv8, x64, out1, W1f, b1r)

    s3 = prop(hs.reshape(2 * NPAD, 32), comb, zeros32)
    out2, u1s = _tc_call(
        _stage_bd_body,
        [_plane_lo(), _plane_hi(), _dlo(), _row_spec(HID), wspec],
        [jax.ShapeDtypeStruct((NPAD, HID), f32),
         jax.ShapeDtypeStruct((NPAD, HID), jnp.bfloat16)],
        [_row_spec(HID), _row_spec(HID)],
    )(s3, s3, dinv8, h1, W2f)

    s4 = prop(u1s.reshape(2 * NPAD, 32), comb, zeros32)
    h2, z = _tc_call(
        _stage_e_body,
        [_plane_lo(), _plane_hi(), _dlo(), _row_spec(HID), _row_spec(HID),
         wspec, bspec, _full_spec((HID, 64)), _full_spec((1, 64)),
         _full_spec((64, 32)), _full_spec((1, 32))],
        [jax.ShapeDtypeStruct((N, HID), f32),
         jax.ShapeDtypeStruct((N, 32), f32)],
        [_row_spec(HID), _row_spec(32)],
    )(s4, s4, dinv8, h1, out2, W2f, b2r,
      Wp1.astype(f32), bp1r, Wp2.astype(f32), bp2r)

    return (h2, z)


# speedup vs baseline: 1.0383x; 1.0383x over previous
import jax
import jax.numpy as jnp
from jax import lax
from jax.experimental import pallas as pl
from jax.experimental.pallas import tpu as pltpu
from jax.experimental.pallas import tpu_sc as plsc

N = 50000
E = 800000
HID = 64

NPAD = 50176
EPAD = 802816
Q = 50176
NCH = 392
CH = 128
ROWS_T = 3136
BN = 3584
GRID = 14
EQ = 1568
ECH = 112
BB = 14
NBB = 28
NBUF = 4

_mesh = lambda: plsc.VectorSubcoreMesh(core_axis_name="c", subcore_axis_name="s")
_SC_PARAMS = pltpu.CompilerParams(use_tc_tiling_on_sc=False)


def _deg_body(src_hbm, ones_hbm, zeros_hbm, out_hbm, idx_v, ones_v, z_v, acc, sem):
    c = lax.axis_index("c")
    s = lax.axis_index("s")
    pltpu.sync_copy(src_hbm.at[pl.ds(s * NCH, NCH)], idx_v)
    pltpu.sync_copy(ones_hbm, ones_v)
    pltpu.sync_copy(zeros_hbm, z_v)
    base = s * ROWS_T
    for k in range(24):
        pltpu.sync_copy(z_v, acc.at[pl.ds(base + k * CH, CH)])
    pltpu.sync_copy(z_v.at[pl.ds(0, 64)], acc.at[pl.ds(base + 24 * CH, 64)])
    plsc.subcore_barrier()
    lo = c * 196
    hi = 196 + c * (NCH - 196)

    def step(j, carry):
        pltpu.sync_copy(ones_v, acc.at[idx_v.at[j]], add=True)
        return carry

    lax.fori_loop(lo, hi, step, 0)
    plsc.subcore_barrier()
    pltpu.sync_copy(acc.at[pl.ds(base, ROWS_T)],
                    out_hbm.at[pl.ds(c * NPAD + base, ROWS_T)])


def _embed_body(a0, a1, a2, a3, t0, t1, t2, t3, o0, o1, o2, o3,
                i0, i1, i2, i3, r0, r1, r2, r3, sem):
    c = lax.axis_index("c")
    s = lax.axis_index("s")
    wid = s * 2 + c
    base = wid * EQ

    idxs = (i0, i1, i2, i3)
    rows = (r0, r1, r2, r3)
    tabs = (t0, t1, t2, t3)
    outs = (o0, o1, o2, o3)
    attrs = (a0, a1, a2, a3)

    def step(k, carry):
        b = base + k * ECH
        for j in range(4):
            pltpu.sync_copy(attrs[j].at[pl.ds(b, ECH)], idxs[j])
            pltpu.async_copy(tabs[j].at[idxs[j]], rows[j], sem).wait()
            pltpu.sync_copy(rows[j], outs[j].at[pl.ds(b, ECH)])
        return carry

    lax.fori_loop(0, 14, step, 0)


def _prop_body(xs_hbm, comb_hbm, zeros_hbm, out_hbm,
               ib0, ib1, r0, r1, r2, r3, acc,
               is0, is1, s0, s1, s2, s3):
    c = lax.axis_index("c")
    s = lax.axis_index("s")
    rows = (r0, r1, r2, r3)
    sems = (s0, s1, s2, s3)
    pltpu.sync_copy(zeros_hbm, r0)
    base = s * ROWS_T
    for k in range(24):
        pltpu.sync_copy(r0, acc.at[pl.ds(base + k * CH, CH)])
    pltpu.sync_copy(r0.at[pl.ds(0, 64)], acc.at[pl.ds(base + 24 * CH, 64)])
    plsc.subcore_barrier()
    cbase = (c * 16 + s) * NBB

    pltpu.async_copy(comb_hbm.at[cbase], ib0, is0)

    def inner(bb, cur, nxt, cursem, nxtsem):
        pltpu.make_async_copy(comb_hbm.at[cbase + bb], cur, cursem).wait()

        @pl.when(bb + 1 < NBB)
        def _():
            pltpu.async_copy(comb_hbm.at[cbase + bb + 1], nxt, nxtsem)

        hs = [None] * NBUF
        for j in range(NBUF):
            hs[j] = pltpu.async_copy(xs_hbm.at[cur.at[j]], rows[j], sems[j])
        for j in range(BB):
            hs[j % NBUF].wait()
            pltpu.sync_copy(rows[j % NBUF], acc.at[cur.at[BB + j]], add=True)
            nxt_j = j + NBUF
            if nxt_j < BB:
                hs[nxt_j % NBUF] = pltpu.async_copy(
                    xs_hbm.at[cur.at[nxt_j]], rows[nxt_j % NBUF],
                    sems[nxt_j % NBUF])

    def blk(bb, carry):
        @pl.when(lax.rem(bb, 2) == 0)
        def _():
            inner(bb, ib0, ib1, is0, is1)

        @pl.when(lax.rem(bb, 2) == 1)
        def _():
            inner(bb, ib1, ib0, is1, is0)

        return carry

    lax.fori_loop(0, NBB, blk, 0)
    plsc.subcore_barrier()
    pltpu.sync_copy(acc.at[pl.ds(base, ROWS_T)],
                    out_hbm.at[pl.ds(c * NPAD + base, ROWS_T)])


def _make_deg():
    return pl.kernel(
        _deg_body,
        out_type=jax.ShapeDtypeStruct((2 * NPAD, 8), jnp.float32),
        mesh=_mesh(),
        scratch_types=[
            pltpu.VMEM((NCH, CH), jnp.int32),
            pltpu.VMEM((CH, 8), jnp.float32),
            pltpu.VMEM((CH, 8), jnp.float32),
            pltpu.VMEM_SHARED((NPAD, 8), jnp.float32),
            pltpu.SemaphoreType.DMA,
        ],
        compiler_params=_SC_PARAMS,
    )


def _make_embed():
    return pl.kernel(
        _embed_body,
        out_type=[jax.ShapeDtypeStruct((NPAD, 8), jnp.float32)] * 4,
        mesh=_mesh(),
        scratch_types=[pltpu.VMEM((ECH,), jnp.int32)] * 4
        + [pltpu.VMEM((ECH, 8), jnp.float32)] * 4
        + [pltpu.SemaphoreType.DMA],
        compiler_params=_SC_PARAMS,
    )


def _make_prop():
    return pl.kernel(
        _prop_body,
        out_type=jax.ShapeDtypeStruct((2 * NPAD, 32), jnp.bfloat16),
        mesh=_mesh(),
        scratch_types=[
            pltpu.VMEM((2 * BB, CH), jnp.int32),
            pltpu.VMEM((2 * BB, CH), jnp.int32),
        ]
        + [pltpu.VMEM((CH, 32), jnp.bfloat16)] * NBUF
        + [pltpu.VMEM_SHARED((NPAD, 32), jnp.bfloat16)]
        + [pltpu.SemaphoreType.DMA] * (2 + NBUF),
        compiler_params=_SC_PARAMS,
    )


def _plane_lo():
    return pl.BlockSpec((BN, 32), lambda i: (i, 0))


def _plane_hi():
    return pl.BlockSpec((BN, 32), lambda i: (i + GRID, 0))


def _row_spec(cols):
    return pl.BlockSpec((BN, cols), lambda i: (i, 0))


def _dlo():
    return pl.BlockSpec((BN, 8), lambda i: (i, 0))


def _dhi():
    return pl.BlockSpec((BN, 8), lambda i: (i + GRID, 0))


def _full_spec(shape):
    nd = len(shape)
    return pl.BlockSpec(shape, lambda i: (0,) * nd)


def _stage_a_body(dlo, dhi, f0, f1, f2, f3, vis, xs_o, x_o, dinv8_o):
    deg = dlo[:, 0:1] + dhi[:, 0:1]
    dinv = jnp.where(deg > 0, lax.rsqrt(jnp.where(deg > 0, deg, 1.0)), 0.0)
    x = jnp.concatenate([f0[...], f1[...], f2[...], f3[...], vis[...]], axis=1)
    x_o[...] = x
    xs_o[...] = (dinv * x).astype(jnp.bfloat16)
    dinv8_o[...] = jnp.broadcast_to(dinv, (BN, 8))


def _stage_bd_body(slo, shi, dinv8, x, W, out_o, ts_o):
    dinv = dinv8[:, 0:1]
    t1 = -dinv * jnp.concatenate([slo[...], shi[...]], axis=1).astype(jnp.float32)
    out_o[...] = (jnp.dot(x[...], W[0], preferred_element_type=jnp.float32)
                  + jnp.dot(t1, W[1], preferred_element_type=jnp.float32))
    ts_o[...] = (dinv * t1).astype(jnp.bfloat16)


def _stage_c_body(slo, shi, dinv8, x, out1, W, b, h_o, hs_o):
    dinv = dinv8[:, 0:1]
    t2 = (-2.0 * dinv * jnp.concatenate([slo[...], shi[...]], axis=1).astype(jnp.float32) - x[...])
    h = jnp.maximum(
        out1[...] + jnp.dot(t2, W[2], preferred_element_type=jnp.float32)
        + b[...], 0.0)
    h_o[...] = h
    hs_o[...] = (dinv * h).astype(jnp.bfloat16)


def _stage_e_body(slo, shi, dinv8, h, out2, W, b, Wp1, bp1, Wp2, bp2,
                  h2_o, z_o):
    dinv = dinv8[:, 0:1]
    t2 = (-2.0 * dinv * jnp.concatenate([slo[...], shi[...]], axis=1).astype(jnp.float32) - h[...])
    h2 = jnp.maximum(
        out2[...] + jnp.dot(t2, W[2], preferred_element_type=jnp.float32)
        + b[...], 0.0)
    h2_o[...] = h2
    a = jnp.dot(h2, Wp1[...], preferred_element_type=jnp.float32) + bp1[...]
    a = jnp.where(a > 0, a, jnp.exp(jnp.minimum(a, 0.0)) - 1.0)
    z_o[...] = jnp.dot(a, Wp2[...], preferred_element_type=jnp.float32) + bp2[...]


def _tc_call(body, in_specs, out_shapes, out_specs):
    return pl.pallas_call(
        body,
        grid=(GRID,),
        in_specs=in_specs,
        out_specs=out_specs,
        out_shape=out_shapes,
    )


@jax.jit
def kernel(edge_index, node_attr, node_vis_feat, len_emb, id_emb, lng_emb,
           lat_emb, W1, b1, W2, b2, Wp1, bp1, Wp2, bp2):
    f32 = jnp.float32
    i32 = jnp.int32

    src = edge_index[0].astype(i32)
    dst = edge_index[1].astype(i32)
    padE = EPAD - E
    srcp = jnp.concatenate([src, jnp.full((padE,), N, i32)])
    dstp = jnp.concatenate([dst, jnp.full((padE,), N, i32)])
    src_plain = srcp.reshape(16 * NCH, CH)
    srco_int = jnp.concatenate([2 * srcp, 2 * srcp + 1]).reshape(2 * 16 * NCH, CH)
    srcb = srco_int.reshape(2, 16, NBB, BB, CH)
    dstb = jnp.broadcast_to(dstp.reshape(1, 16, NBB, BB, CH),
                            (2, 16, NBB, BB, CH))
    comb = jnp.concatenate([srcb, dstb], axis=3).reshape(
        2 * 16 * NBB, 2 * BB, CH)

    attr = jnp.pad(node_attr.astype(i32), ((0, NPAD - N), (0, 0)))
    vis_p = jnp.pad(node_vis_feat.astype(f32), ((0, NPAD - N), (0, 0)))

    zeros8 = jnp.zeros((CH, 8), f32)
    ones8 = jnp.ones((CH, 8), f32)
    zeros32 = jnp.zeros((CH, 32), jnp.bfloat16)

    deg2 = _make_deg()(src_plain, ones8, zeros8)
    f_id, f_len, f_lng, f_lat = _make_embed()(
        attr[:, 1], attr[:, 0], attr[:, 2], attr[:, 3],
        id_emb.astype(f32), len_emb.astype(f32),
        lng_emb.astype(f32), lat_emb.astype(f32))

    prop = _make_prop()

    b1r = b1.reshape(1, HID).astype(f32)
    b2r = b2.reshape(1, HID).astype(f32)
    bp1r = bp1.reshape(1, 64).astype(f32)
    bp2r = bp2.reshape(1, 32).astype(f32)
    W1f = W1.astype(f32)
    W2f = W2.astype(f32)

    wspec = _full_spec((3, HID, HID))
    bspec = _full_spec((1, HID))

    xs, x64, dinv8 = _tc_call(
        _stage_a_body,
        [_dlo(), _dhi(), _dlo(), _dlo(), _dlo(), _dlo(), _row_spec(32)],
        [jax.ShapeDtypeStruct((NPAD, HID), jnp.bfloat16),
         jax.ShapeDtypeStruct((NPAD, HID), f32),
         jax.ShapeDtypeStruct((NPAD, 8), f32)],
        [_row_spec(HID), _row_spec(HID), _dlo()],
    )(deg2, deg2, f_id, f_len, f_lng, f_lat, vis_p)

    s1 = prop(xs.reshape(2 * NPAD, 32), comb, zeros32)
    out1, t1s = _tc_call(
        _stage_bd_body,
        [_plane_lo(), _plane_hi(), _dlo(), _row_spec(HID), wspec],
        [jax.ShapeDtypeStruct((NPAD, HID), f32),
         jax.ShapeDtypeStruct((NPAD, HID), jnp.bfloat16)],
        [_row_spec(HID), _row_spec(HID)],
    )(s1, s1, dinv8, x64, W1f)

    s2 = prop(t1s.reshape(2 * NPAD, 32), comb, zeros32)
    h1, hs = _tc_call(
        _stage_c_body,
        [_plane_lo(), _plane_hi(), _dlo(), _row_spec(HID), _row_spec(HID),
         wspec, bspec],
        [jax.ShapeDtypeStruct((NPAD, HID), f32),
         jax.ShapeDtypeStruct((NPAD, HID), jnp.bfloat16)],
        [_row_spec(HID), _row_spec(HID)],
    )(s2, s2, dinv8, x64, out1, W1f, b1r)

    s3 = prop(hs.reshape(2 * NPAD, 32), comb, zeros32)
    out2, u1s = _tc_call(
        _stage_bd_body,
        [_plane_lo(), _plane_hi(), _dlo(), _row_spec(HID), wspec],
        [jax.ShapeDtypeStruct((NPAD, HID), f32),
         jax.ShapeDtypeStruct((NPAD, HID), jnp.bfloat16)],
        [_row_spec(HID), _row_spec(HID)],
    )(s3, s3, dinv8, h1, W2f)

    s4 = prop(u1s.reshape(2 * NPAD, 32), comb, zeros32)
    h2, z = _tc_call(
        _stage_e_body,
        [_plane_lo(), _plane_hi(), _dlo(), _row_spec(HID), _row_spec(HID),
         wspec, bspec, _full_spec((HID, 64)), _full_spec((1, 64)),
         _full_spec((64, 32)), _full_spec((1, 32))],
        [jax.ShapeDtypeStruct((N, HID), f32),
         jax.ShapeDtypeStruct((N, 32), f32)],
        [_row_spec(HID), _row_spec(32)],
    )(s4, s4, dinv8, h1, out2, W2f, b2r,
      Wp1.astype(f32), bp1r, Wp2.astype(f32), bp2r)

    return (h2, z)
